# baseline (speedup 1.0000x reference)
import functools

import jax
import jax.numpy as jnp
from jax import lax
from jax.experimental import pallas as pl
from jax.experimental.pallas import tpu as pltpu
from jax.experimental.pallas import tpu_sc as plsc

N_USER = 25000
N_ITEM = 25000
N = N_USER + N_ITEM
E = 800000
D = 64
BATCH = 1024

NC = 2
NS = 16
NW = NC * NS
HALF = N // NC
ACC_ROWS = 25008
CH = 128
BLK = 2048
NCHB = BLK // CH
EPT = 51200
NBLK = EPT // BLK
E_PAD = EPT * NS
STG = 4224

_mesh = plsc.VectorSubcoreMesh(core_axis_name="c", subcore_axis_name="s")
_cparams = pltpu.CompilerParams(use_tc_tiling_on_sc=False,
                                needs_layout_passes=False)


def _partition_body(rows_hbm, cols_hbm, vals_hbm,
                    plrows, pcols, pvals, counts,
                    rowsv, colsv, valsv, srow, scol, sval, cntv, sem):
    c = lax.axis_index("c")
    s = lax.axis_index("s")
    wid = s * NC + c
    base = c * jnp.int32(HALF)
    wslot = wid * (EPT // CH)

    def block_body(b, carry):
        off, wpos = carry
        cb = s * (EPT // CH) + b * NCHB
        pltpu.sync_copy(rows_hbm.at[pl.ds(cb, NCHB)], rowsv)
        pltpu.sync_copy(cols_hbm.at[pl.ds(cb, NCHB)], colsv)
        pltpu.sync_copy(vals_hbm.at[pl.ds(cb, NCHB)], valsv)

        def grp(g, off2):
            j = g >> 3
            jj = jnp.bitwise_and(g, 7) * 16
            r = rowsv[j, pl.ds(jj, 16)]
            cl = colsv[j, pl.ds(jj, 16)]
            v = valsv[j, pl.ds(jj, 16)]
            local = r - base
            msk = (local >= 0) & (local < HALF)
            plsc.store_compressed(srow.at[pl.ds(off2, 16)], local, mask=msk)
            plsc.store_compressed(scol.at[pl.ds(off2, 16)], cl, mask=msk)
            plsc.store_compressed(sval.at[pl.ds(off2, 16)], v, mask=msk)
            return off2 + plsc.all_reduce_population_count(msk)[0]

        off = lax.fori_loop(0, BLK // 16, grp, off)

        flush = off >= BLK

        @pl.when(flush)
        def _():
            pltpu.sync_copy(srow.at[pl.ds(0, BLK)],
                            plrows.at[pl.ds((wslot + wpos) * CH, BLK)])
            pltpu.sync_copy(scol.at[pl.ds(0, BLK)],
                            pcols.at[pl.ds((wslot + wpos) * CH, BLK)])
            pltpu.sync_copy(sval.at[pl.ds(0, BLK)],
                            pvals.at[pl.ds((wslot + wpos) * CH, BLK)])

            def mv(g, carry2):
                sl_src = pl.ds(BLK + g * 16, 16)
                sl_dst = pl.ds(g * 16, 16)
                srow[sl_dst] = srow[sl_src]
                scol[sl_dst] = scol[sl_src]
                sval[sl_dst] = sval[sl_src]
                return carry2
            lax.fori_loop(0, (STG - BLK) // 16, mv, 0)

        off = jnp.where(flush, off - BLK, off)
        wpos = jnp.where(flush, wpos + NCHB, wpos)
        return (off, wpos)

    off, wpos = lax.fori_loop(0, NBLK, block_body,
                              (jnp.int32(0), jnp.int32(0)))

    iota = lax.broadcasted_iota(jnp.int32, (16,), 0)

    def fill(g, carry):
        pos = iota + g * 16
        m = pos >= off
        sl = pl.ds(g * 16, 16)
        srow[sl] = jnp.where(m, jnp.int32(HALF), srow[sl])
        scol[sl] = jnp.where(m, jnp.int32(0), scol[sl])
        sval[sl] = jnp.where(m, jnp.float32(0), sval[sl])
        return carry

    lax.fori_loop(0, STG // 16, fill, 0)

    nrem = (off + BLK - 1) >> 11

    @pl.when(nrem >= 1)
    def _():
        pltpu.sync_copy(srow.at[pl.ds(0, BLK)],
                        plrows.at[pl.ds((wslot + wpos) * CH, BLK)])
        pltpu.sync_copy(scol.at[pl.ds(0, BLK)],
                        pcols.at[pl.ds((wslot + wpos) * CH, BLK)])
        pltpu.sync_copy(sval.at[pl.ds(0, BLK)],
                        pvals.at[pl.ds((wslot + wpos) * CH, BLK)])

    @pl.when(nrem >= 2)
    def _():
        pltpu.sync_copy(srow.at[pl.ds(BLK, BLK)],
                        plrows.at[pl.ds((wslot + wpos + NCHB) * CH, BLK)])
        pltpu.sync_copy(scol.at[pl.ds(BLK, BLK)],
                        pcols.at[pl.ds((wslot + wpos + NCHB) * CH, BLK)])
        pltpu.sync_copy(sval.at[pl.ds(BLK, BLK)],
                        pvals.at[pl.ds((wslot + wpos + NCHB) * CH, BLK)])

    nblk_out = (wpos >> 4) + nrem
    cntv[...] = jnp.broadcast_to(nblk_out, (16,))
    pltpu.sync_copy(cntv, counts.at[wid])


_partition = functools.partial(
    pl.kernel,
    out_type=[
        jax.ShapeDtypeStruct((NW * EPT,), jnp.int32),
        jax.ShapeDtypeStruct((NW * EPT,), jnp.int32),
        jax.ShapeDtypeStruct((NW * EPT,), jnp.float32),
        jax.ShapeDtypeStruct((NW, 16), jnp.int32),
    ],
    mesh=_mesh,
    scratch_types=[
        pltpu.VMEM((NCHB, CH), jnp.int32),
        pltpu.VMEM((NCHB, CH), jnp.int32),
        pltpu.VMEM((NCHB, CH), jnp.float32),
        pltpu.VMEM((STG,), jnp.int32),
        pltpu.VMEM((STG,), jnp.int32),
        pltpu.VMEM((STG,), jnp.float32),
        pltpu.VMEM((16,), jnp.int32),
        pltpu.SemaphoreType.DMA,
    ],
    compiler_params=_cparams,
)(_partition_body)


NSEG = 25
SLOTC = EPT // CH
STG2 = 416


def _colsplit_body(lrows_hbm, cols_hbm, vals_hbm, cntsA_hbm,
                   qrows, qcols, qvals, counts2,
                   rowsv, colsv, valsv, rstage, cstage, vstage, cntv, c2v):
    c = lax.axis_index("c")
    s = lax.axis_index("s")
    wid = s * NC + c

    pltpu.sync_copy(cntsA_hbm.at[wid], cntv)
    nblkA = cntv[...][0]

    iota = lax.broadcasted_iota(jnp.int32, (16,), 0)
    zero16 = jnp.zeros((16,), jnp.int32)

    def block_body(b, carry):
        offs, wpos = carry
        cb = wid * (EPT // CH) + b * NCHB

        pltpu.sync_copy(lrows_hbm.at[pl.ds(cb, NCHB)], rowsv)
        pltpu.sync_copy(cols_hbm.at[pl.ds(cb, NCHB)], colsv)
        pltpu.sync_copy(vals_hbm.at[pl.ds(cb, NCHB)], valsv)

        def win_body(w, carry2):
            offs2, wpos2 = carry2
            j = w

            def grp(g, offs3):
                gg = g * 16
                lr = rowsv[j, pl.ds(gg, 16)]
                cl = colsv[j, pl.ds(gg, 16)]
                v = valsv[j, pl.ds(gg, 16)]
                seg16 = lax.shift_right_logical(cl, 11)
                lcol = jnp.bitwise_and(cl, 2047)
                valid = lr < HALF
                new_offs = []
                for t in range(NSEG):
                    m = valid & (seg16 == t)
                    o = offs3[t]
                    plsc.store_compressed(rstage.at[t].at[pl.ds(o, 16)],
                                          lr, mask=m)
                    plsc.store_compressed(cstage.at[t].at[pl.ds(o, 16)],
                                          lcol, mask=m)
                    plsc.store_compressed(vstage.at[t].at[pl.ds(o, 16)],
                                          v, mask=m)
                    new_offs.append(
                        o + plsc.all_reduce_population_count(m)[0])
                return tuple(new_offs)

            offs2 = lax.fori_loop(0, 8, grp, offs2)

            new_offs, new_wpos = [], []
            for t in range(NSEG):
                o, wp = offs2[t], wpos2[t]
                full = o >= CH

                @pl.when(full)
                def _(t=t, wp=wp):
                    dst = ((wid * NSEG + t) * SLOTC + wp) * CH
                    pltpu.sync_copy(rstage.at[t].at[pl.ds(0, CH)],
                                    qrows.at[pl.ds(dst, CH)])
                    pltpu.sync_copy(cstage.at[t].at[pl.ds(0, CH)],
                                    qcols.at[pl.ds(dst, CH)])
                    pltpu.sync_copy(vstage.at[t].at[pl.ds(0, CH)],
                                    qvals.at[pl.ds(dst, CH)])

                    def mv(g, carry3):
                        sl_s = pl.ds(CH + g * 16, 16)
                        sl_d = pl.ds(g * 16, 16)
                        rstage[t, sl_d] = rstage[t, sl_s]
                        cstage[t, sl_d] = cstage[t, sl_s]
                        vstage[t, sl_d] = vstage[t, sl_s]
                        return carry3
                    lax.fori_loop(0, (STG2 - CH) // 16, mv, 0)

                new_offs.append(jnp.where(full, o - CH, o))
                new_wpos.append(jnp.where(full, wp + 1, wp))
            return (tuple(new_offs), tuple(new_wpos))

        return lax.fori_loop(0, NCHB, win_body, (offs, wpos))

    z25 = (jnp.int32(0),) * NSEG
    offs, wpos = lax.fori_loop(0, nblkA, block_body, (z25, z25))

    cnt_lanes = [zero16, zero16]
    for t in range(NSEG):
        o = offs[t]

        def fill(g, carry):
            pos = iota + g * 16
            m = pos >= o
            sl = pl.ds(g * 16, 16)
            rstage[t, sl] = jnp.where(m, jnp.int32(HALF), rstage[t, sl])
            cstage[t, sl] = jnp.where(m, zero16, cstage[t, sl])
            vstage[t, sl] = jnp.where(m, jnp.float32(0), vstage[t, sl])
            return carry

        lax.fori_loop(0, CH * 2 // 16, fill, 0)
        nrem = (o + CH - 1) >> 7

        @pl.when(nrem >= 1)
        def _(t=t, wp=wpos[t]):
            dst = ((wid * NSEG + t) * SLOTC + wp) * CH
            pltpu.sync_copy(rstage.at[t].at[pl.ds(0, CH)],
                            qrows.at[pl.ds(dst, CH)])
            pltpu.sync_copy(cstage.at[t].at[pl.ds(0, CH)],
                            qcols.at[pl.ds(dst, CH)])
            pltpu.sync_copy(vstage.at[t].at[pl.ds(0, CH)],
                            qvals.at[pl.ds(dst, CH)])

        @pl.when(nrem >= 2)
        def _(t=t, wp=wpos[t]):
            dst = ((wid * NSEG + t) * SLOTC + wp + 1) * CH
            pltpu.sync_copy(rstage.at[t].at[pl.ds(CH, CH)],
                            qrows.at[pl.ds(dst, CH)])
            pltpu.sync_copy(cstage.at[t].at[pl.ds(CH, CH)],
                            qcols.at[pl.ds(dst, CH)])
            pltpu.sync_copy(vstage.at[t].at[pl.ds(CH, CH)],
                            qvals.at[pl.ds(dst, CH)])

        cnt_t = wpos[t] + nrem
        c2v[...] = jnp.broadcast_to(cnt_t, (16,))
        pltpu.sync_copy(c2v, counts2.at[wid * 32 + t])


_colsplit = functools.partial(
    pl.kernel,
    out_type=[
        jax.ShapeDtypeStruct((NW * NSEG * SLOTC * CH,), jnp.int32),
        jax.ShapeDtypeStruct((NW * NSEG * SLOTC * CH,), jnp.int32),
        jax.ShapeDtypeStruct((NW * NSEG * SLOTC * CH,), jnp.float32),
        jax.ShapeDtypeStruct((NW * 32, 16), jnp.int32),
    ],
    mesh=_mesh,
    scratch_types=[
        pltpu.VMEM((NCHB, CH), jnp.int32),
        pltpu.VMEM((NCHB, CH), jnp.int32),
        pltpu.VMEM((NCHB, CH), jnp.float32),
        pltpu.VMEM((NSEG, STG2), jnp.int32),
        pltpu.VMEM((NSEG, STG2), jnp.int32),
        pltpu.VMEM((NSEG, STG2), jnp.float32),
        pltpu.VMEM((16,), jnp.int32),
        pltpu.VMEM((16,), jnp.int32),
    ],
    compiler_params=_cparams,
)(_colsplit_body)


def _spmm3_body(qrows_hbm, qcols_hbm, qvals_hbm, cnts2_hbm, x_hbm, out_hbm,
                acc, xstage, gbuf, rstg, cstg, vstg, cntv, gsem, ssem):
    c = lax.axis_index("c")
    s = lax.axis_index("s")
    wid = s * NC + c

    zvec = jnp.zeros((16,), jnp.float32)

    def zrow(i, carry):
        for jj in range(4):
            gbuf[0, i, pl.ds(jj * 16, 16)] = zvec
        return carry

    lax.fori_loop(0, CH, zrow, 0)
    rows_per_tile = ACC_ROWS // NS
    tbase = s * rows_per_tile
    for zi in range(rows_per_tile // CH):
        pltpu.sync_copy(gbuf.at[0], acc.at[pl.ds(tbase + zi * CH, CH)])
    rem = rows_per_tile % CH
    if rem:
        pltpu.sync_copy(gbuf.at[0].at[pl.ds(0, rem)],
                        acc.at[pl.ds(tbase + rows_per_tile - rem, rem)])

    plsc.subcore_barrier()

    def gstart(j, buf):
        pltpu.async_copy(xstage.at[cstg.at[j]], gbuf.at[buf], gsem.at[buf])

    def gwait(j, buf):
        pltpu.make_async_copy(xstage.at[cstg.at[j]], gbuf.at[buf],
                              gsem.at[buf]).wait()

    def sstart(j, buf):
        pltpu.async_copy(gbuf.at[buf], acc.at[rstg.at[j]], ssem.at[buf],
                         add=True)

    def swait(j, buf):
        pltpu.make_async_copy(gbuf.at[buf], acc.at[rstg.at[j]],
                              ssem.at[buf]).wait()

    def seg_body(seg, carry0):
        pltpu.sync_copy(x_hbm.at[pl.ds(seg * 2048 + s * CH, CH)],
                        xstage.at[pl.ds(s * CH, CH)])
        pltpu.sync_copy(cnts2_hbm.at[wid * 32 + seg], cntv)
        plsc.subcore_barrier()

        cnt = cntv[...][0]
        bbase = (wid * NSEG + seg) * SLOTC
        nstg = (cnt + 7) >> 3

        def stage_body(st, carry, cnt=cnt, bbase=bbase):
            pltpu.sync_copy(qrows_hbm.at[pl.ds(bbase + st * 8, 8)], rstg)
            pltpu.sync_copy(qcols_hbm.at[pl.ds(bbase + st * 8, 8)], cstg)
            pltpu.sync_copy(qvals_hbm.at[pl.ds(bbase + st * 8, 8)], vstg)
            nch = jnp.minimum(jnp.int32(8), cnt - st * 8)

            gstart(0, 0)

            def chunk_body(j, carry2):
                buf = jnp.bitwise_and(j, 1)
                gwait(j, buf)

                @pl.when(j + 1 < nch)
                def _prefetch():
                    @pl.when(j >= 1)
                    def _():
                        swait(j - 1, 1 - buf)
                    gstart(j + 1, 1 - buf)

                @plsc.parallel_loop(0, CH // 16, unroll=4)
                def scale_body(g):
                    e0 = g * 16
                    vals16 = vstg[j, pl.ds(e0, 16)]
                    for e2 in range(16):
                        e = e0 + e2
                        vsplat = jnp.broadcast_to(vals16[e2], (16,))
                        for kk in range(4):
                            sl = pl.ds(kk * 16, 16)
                            gbuf[buf, e, sl] = gbuf[buf, e, sl] * vsplat

                sstart(j, buf)
                return carry2

            lax.fori_loop(0, nch, chunk_body, 0)

            @pl.when(nch >= 2)
            def _():
                swait(nch - 2, jnp.bitwise_and(nch - 2, 1))
            swait(nch - 1, jnp.bitwise_and(nch - 1, 1))
            return carry

        lax.fori_loop(0, nstg, stage_body, 0)
        plsc.subcore_barrier()
        return carry0

    lax.fori_loop(0, NSEG, seg_body, 0)

    WB = 200
    NWB = HALF // WB

    def wb_body(w, carry):
        idx = s + w * NS

        @pl.when(idx < NWB)
        def _():
            pltpu.sync_copy(acc.at[pl.ds(idx * WB, WB)],
                            out_hbm.at[pl.ds(c * HALF + idx * WB, WB)])
        return carry

    lax.fori_loop(0, (NWB + NS - 1) // NS, wb_body, 0)


_spmm3 = functools.partial(
    pl.kernel,
    out_type=jax.ShapeDtypeStruct((N, D), jnp.float32),
    mesh=_mesh,
    scratch_types=[
        pltpu.VMEM_SHARED((ACC_ROWS, D), jnp.float32),
        pltpu.VMEM_SHARED((2048, D), jnp.float32),
        pltpu.VMEM((2, CH, D), jnp.float32),
        pltpu.VMEM((8, CH), jnp.int32),
        pltpu.VMEM((8, CH), jnp.int32),
        pltpu.VMEM((8, CH), jnp.float32),
        pltpu.VMEM((16,), jnp.int32),
        pltpu.SemaphoreType.DMA((2,)),
        pltpu.SemaphoreType.DMA((2,)),
    ],
    compiler_params=_cparams,
)(_spmm3_body)


def _spmm_body(lrows_hbm, cols_hbm, vals_hbm, cnts_hbm, x_hbm, out_hbm,
               acc, gbuf, lidxv, colsv, valsv, cntv, gsem, ssem):
    c = lax.axis_index("c")
    s = lax.axis_index("s")
    wid = s * NC + c

    zvec = jnp.zeros((16,), jnp.float32)

    def zrow(i, carry):
        for jj in range(4):
            gbuf[0, i, pl.ds(jj * 16, 16)] = zvec
        return carry

    lax.fori_loop(0, CH, zrow, 0)
    rows_per_tile = ACC_ROWS // NS
    tbase = s * rows_per_tile
    for zi in range(rows_per_tile // CH):
        pltpu.sync_copy(gbuf.at[0], acc.at[pl.ds(tbase + zi * CH, CH)])
    rem = rows_per_tile % CH
    if rem:
        pltpu.sync_copy(gbuf.at[0].at[pl.ds(0, rem)],
                        acc.at[pl.ds(tbase + rows_per_tile - rem, rem)])

    pltpu.sync_copy(cnts_hbm.at[wid], cntv)
    nblk = cntv[...][0]
    plsc.subcore_barrier()

    NB = 3

    def gstart(j, buf):
        pltpu.async_copy(x_hbm.at[colsv.at[j]], gbuf.at[buf], gsem.at[buf])

    def gwait(j, buf):
        pltpu.make_async_copy(x_hbm.at[colsv.at[j]], gbuf.at[buf],
                              gsem.at[buf]).wait()

    def sstart(j, buf):
        pltpu.async_copy(gbuf.at[buf], acc.at[lidxv.at[j]], ssem.at[buf],
                         add=True)

    def swait(j, buf):
        pltpu.make_async_copy(gbuf.at[buf], acc.at[lidxv.at[j]],
                              ssem.at[buf]).wait()

    def block_body(b, carry):
        run = b < nblk

        @pl.when(run)
        def _blk():
            cb = wid * (EPT // CH) + b * NCHB
            pltpu.sync_copy(lrows_hbm.at[pl.ds(cb, NCHB)], lidxv)
            pltpu.sync_copy(cols_hbm.at[pl.ds(cb, NCHB)], colsv)
            pltpu.sync_copy(vals_hbm.at[pl.ds(cb, NCHB)], valsv)

            gstart(0, 0)
            gstart(1, 1)

            def chunk_body(j, carry2):
                buf = jnp.remainder(j, NB)
                gwait(j, buf)

                def scale_body(g, carry3):
                    e0 = g * 16
                    vals16 = valsv[j, pl.ds(e0, 16)]
                    for e2 in range(16):
                        e = e0 + e2
                        vsplat = jnp.broadcast_to(vals16[e2], (16,))
                        for kk in range(4):
                            sl = pl.ds(kk * 16, 16)
                            gbuf[buf, e, sl] = gbuf[buf, e, sl] * vsplat
                    return carry3

                lax.fori_loop(0, CH // 16, scale_body, 0)
                sstart(j, buf)

                @pl.when(j + 2 < NCHB)
                def _prefetch():
                    nbuf = jnp.remainder(j + 2, NB)

                    @pl.when(j >= 1)
                    def _():
                        swait(j - 1, nbuf)
                    gstart(j + 2, nbuf)
                return carry2

            lax.fori_loop(0, NCHB, chunk_body, 0)
            swait(NCHB - 3, jnp.remainder(NCHB - 3, NB))
            swait(NCHB - 2, jnp.remainder(NCHB - 2, NB))
            swait(NCHB - 1, jnp.remainder(NCHB - 1, NB))
        return carry

    lax.fori_loop(0, NBLK, block_body, 0)
    plsc.subcore_barrier()

    WB = 200
    NWB = HALF // WB

    def wb_body(w, carry):
        idx = s + w * NS

        @pl.when(idx < NWB)
        def _():
            pltpu.sync_copy(acc.at[pl.ds(idx * WB, WB)],
                            out_hbm.at[pl.ds(c * HALF + idx * WB, WB)])
        return carry

    lax.fori_loop(0, (NWB + NS - 1) // NS, wb_body, 0)


_spmm = functools.partial(
    pl.kernel,
    out_type=jax.ShapeDtypeStruct((N, D), jnp.float32),
    mesh=_mesh,
    scratch_types=[
        pltpu.VMEM_SHARED((ACC_ROWS, D), jnp.float32),
        pltpu.VMEM((3, CH, D), jnp.float32),
        pltpu.VMEM((NCHB, CH), jnp.int32),
        pltpu.VMEM((NCHB, CH), jnp.int32),
        pltpu.VMEM((NCHB, CH), jnp.float32),
        pltpu.VMEM((16,), jnp.int32),
        pltpu.SemaphoreType.DMA((3,)),
        pltpu.SemaphoreType.DMA((3,)),
    ],
    compiler_params=_cparams,
)(_spmm_body)


def _dense_body(L_ref, e_ref, wg_ref, wb_ref, b_ref, act_ref, nrm_ref):
    l = L_ref[...]
    e = e_ref[...]
    x = (jnp.dot(l + e, wg_ref[...], preferred_element_type=jnp.float32)
         + jnp.dot(l * e, wb_ref[...], preferred_element_type=jnp.float32)
         + 2.0 * b_ref[...])
    act = jnp.where(x >= 0, x, 0.01 * x)
    act_ref[...] = act
    nrm = jnp.maximum(jnp.sqrt(jnp.sum(act * act, axis=1, keepdims=True)),
                      1e-12)
    nrm_ref[...] = act / nrm


_DENSE_R = 1000


def _dense(L, ego, Wg, Wb, b):
    return pl.pallas_call(
        _dense_body,
        grid=(N // _DENSE_R,),
        in_specs=[
            pl.BlockSpec((_DENSE_R, D), lambda i: (i, 0)),
            pl.BlockSpec((_DENSE_R, D), lambda i: (i, 0)),
            pl.BlockSpec((D, D), lambda i: (0, 0)),
            pl.BlockSpec((D, D), lambda i: (0, 0)),
            pl.BlockSpec((1, D), lambda i: (0, 0)),
        ],
        out_specs=[
            pl.BlockSpec((_DENSE_R, D), lambda i: (i, 0)),
            pl.BlockSpec((_DENSE_R, D), lambda i: (i, 0)),
        ],
        out_shape=[jax.ShapeDtypeStruct((N, D), jnp.float32)] * 2,
    )(L, ego, Wg, Wb, b)


GP = BATCH // (NC * NS)


def _final_gather_body(t0, t1, t2, t3, users, pos, neg, *rest):
    outs = rest[:12]
    idxv, rowsv, sem = rest[12:]
    wid = lax.axis_index("s") * NC + lax.axis_index("c")
    gbase = wid * GP
    tabs = (t0, t1, t2, t3)
    for bi, bidx in enumerate((users, pos, neg)):
        pltpu.sync_copy(bidx.at[pl.ds(gbase, GP)], idxv)
        if bi > 0:
            for jj in range(GP // 16):
                sl = pl.ds(jj * 16, 16)
                idxv[sl] = idxv[sl] + jnp.int32(N_USER)
        for ti in range(4):
            pltpu.async_copy(tabs[ti].at[idxv], rowsv, sem).wait()
            pltpu.sync_copy(rowsv, outs[bi * 4 + ti].at[pl.ds(gbase, GP)])


_final_gather = functools.partial(
    pl.kernel,
    out_type=[jax.ShapeDtypeStruct((BATCH, D), jnp.float32)] * 12,
    mesh=_mesh,
    scratch_types=[
        pltpu.VMEM((GP,), jnp.int32),
        pltpu.VMEM((GP, D), jnp.float32),
        pltpu.SemaphoreType.DMA,
    ],
    compiler_params=pltpu.CompilerParams(use_tc_tiling_on_sc=False),
)(_final_gather_body)


def kernel(user_emb, item_emb, W_gc_0, W_gc_1, W_gc_2, W_bi_0, W_bi_1, W_bi_2,
           b_bi_0, b_bi_1, b_bi_2, edge_index, edge_vals, users, pos_items,
           neg_items):
    ego0 = jnp.concatenate([user_emb, item_emb], axis=0)
    pad = E_PAD - E
    rows2d = jnp.concatenate(
        [edge_index[0], jnp.full((pad,), N, jnp.int32)]).reshape(E_PAD // CH, CH)
    cols2d = jnp.concatenate(
        [edge_index[1], jnp.zeros((pad,), jnp.int32)]).reshape(E_PAD // CH, CH)
    vals2d = jnp.concatenate(
        [edge_vals, jnp.zeros((pad,), jnp.float32)]).reshape(E_PAD // CH, CH)

    plr, plc, plv, cnts = _partition(rows2d, cols2d, vals2d)
    plr = plr.reshape(NW * EPT // CH, CH)
    plc = plc.reshape(NW * EPT // CH, CH)
    plv = plv.reshape(NW * EPT // CH, CH)

    qr, qc, qv, cnts2 = _colsplit(plr, plc, plv, cnts)
    qr = qr.reshape(NW * NSEG * SLOTC, CH)
    qc = qc.reshape(NW * NSEG * SLOTC, CH)
    qv = qv.reshape(NW * NSEG * SLOTC, CH)

    Ws_gc = (W_gc_0, W_gc_1, W_gc_2)
    Ws_bi = (W_bi_0, W_bi_1, W_bi_2)
    bs = (b_bi_0, b_bi_1, b_bi_2)

    xpad = jnp.zeros((NSEG * 2048 - N, D), jnp.float32)
    ego = ego0
    norms = []
    for k in range(3):
        L = _spmm3(qr, qc, qv, cnts2, jnp.concatenate([ego, xpad], axis=0))
        ego, nrm = _dense(L, ego, Ws_gc[k], Ws_bi[k], bs[k])
        norms.append(nrm)

    outs = _final_gather(ego0, norms[0], norms[1], norms[2],
                         users, pos_items, neg_items)
    u = jnp.concatenate(outs[0:4], axis=1)
    p = jnp.concatenate(outs[4:8], axis=1)
    ng = jnp.concatenate(outs[8:12], axis=1)
    return u, p, ng


# speedup vs baseline: 1.0692x; 1.0692x over previous
import functools

import jax
import jax.numpy as jnp
from jax import lax
from jax.experimental import pallas as pl
from jax.experimental.pallas import tpu as pltpu
from jax.experimental.pallas import tpu_sc as plsc

N_USER = 25000
N_ITEM = 25000
N = N_USER + N_ITEM
E = 800000
D = 64
BATCH = 1024

NC = 2
NS = 16
NW = NC * NS
HALF = N // NC
ACC_ROWS = 25008
CH = 128
BLK = 2048
NCHB = BLK // CH
EPT = 51200
NBLK = EPT // BLK
E_PAD = EPT * NS
STG = 4224

_mesh = plsc.VectorSubcoreMesh(core_axis_name="c", subcore_axis_name="s")
_cparams = pltpu.CompilerParams(use_tc_tiling_on_sc=False,
                                needs_layout_passes=False)


def _partition_body(rows_hbm, cols_hbm, vals_hbm,
                    plrows, pcols, pvals, counts,
                    rowsv, colsv, valsv, srow, scol, sval, cntv, sem):
    c = lax.axis_index("c")
    s = lax.axis_index("s")
    wid = s * NC + c
    base = c * jnp.int32(HALF)
    wslot = wid * (EPT // CH)

    def block_body(b, carry):
        off, wpos = carry
        cb = s * (EPT // CH) + b * NCHB
        pltpu.sync_copy(rows_hbm.at[pl.ds(cb, NCHB)], rowsv)
        pltpu.sync_copy(cols_hbm.at[pl.ds(cb, NCHB)], colsv)
        pltpu.sync_copy(vals_hbm.at[pl.ds(cb, NCHB)], valsv)

        def grp(g, off2):
            j = g >> 3
            jj = jnp.bitwise_and(g, 7) * 16
            r = rowsv[j, pl.ds(jj, 16)]
            cl = colsv[j, pl.ds(jj, 16)]
            v = valsv[j, pl.ds(jj, 16)]
            local = r - base
            msk = (local >= 0) & (local < HALF)
            plsc.store_compressed(srow.at[pl.ds(off2, 16)], local, mask=msk)
            plsc.store_compressed(scol.at[pl.ds(off2, 16)], cl, mask=msk)
            plsc.store_compressed(sval.at[pl.ds(off2, 16)], v, mask=msk)
            return off2 + plsc.all_reduce_population_count(msk)[0]

        off = lax.fori_loop(0, BLK // 16, grp, off)

        flush = off >= BLK

        @pl.when(flush)
        def _():
            pltpu.sync_copy(srow.at[pl.ds(0, BLK)],
                            plrows.at[pl.ds((wslot + wpos) * CH, BLK)])
            pltpu.sync_copy(scol.at[pl.ds(0, BLK)],
                            pcols.at[pl.ds((wslot + wpos) * CH, BLK)])
            pltpu.sync_copy(sval.at[pl.ds(0, BLK)],
                            pvals.at[pl.ds((wslot + wpos) * CH, BLK)])

            def mv(g, carry2):
                sl_src = pl.ds(BLK + g * 16, 16)
                sl_dst = pl.ds(g * 16, 16)
                srow[sl_dst] = srow[sl_src]
                scol[sl_dst] = scol[sl_src]
                sval[sl_dst] = sval[sl_src]
                return carry2
            lax.fori_loop(0, (STG - BLK) // 16, mv, 0)

        off = jnp.where(flush, off - BLK, off)
        wpos = jnp.where(flush, wpos + NCHB, wpos)
        return (off, wpos)

    off, wpos = lax.fori_loop(0, NBLK, block_body,
                              (jnp.int32(0), jnp.int32(0)))

    iota = lax.broadcasted_iota(jnp.int32, (16,), 0)

    def fill(g, carry):
        pos = iota + g * 16
        m = pos >= off
        sl = pl.ds(g * 16, 16)
        srow[sl] = jnp.where(m, jnp.int32(HALF), srow[sl])
        scol[sl] = jnp.where(m, jnp.int32(0), scol[sl])
        sval[sl] = jnp.where(m, jnp.float32(0), sval[sl])
        return carry

    lax.fori_loop(0, STG // 16, fill, 0)

    nrem = (off + BLK - 1) >> 11

    @pl.when(nrem >= 1)
    def _():
        pltpu.sync_copy(srow.at[pl.ds(0, BLK)],
                        plrows.at[pl.ds((wslot + wpos) * CH, BLK)])
        pltpu.sync_copy(scol.at[pl.ds(0, BLK)],
                        pcols.at[pl.ds((wslot + wpos) * CH, BLK)])
        pltpu.sync_copy(sval.at[pl.ds(0, BLK)],
                        pvals.at[pl.ds((wslot + wpos) * CH, BLK)])

    @pl.when(nrem >= 2)
    def _():
        pltpu.sync_copy(srow.at[pl.ds(BLK, BLK)],
                        plrows.at[pl.ds((wslot + wpos + NCHB) * CH, BLK)])
        pltpu.sync_copy(scol.at[pl.ds(BLK, BLK)],
                        pcols.at[pl.ds((wslot + wpos + NCHB) * CH, BLK)])
        pltpu.sync_copy(sval.at[pl.ds(BLK, BLK)],
                        pvals.at[pl.ds((wslot + wpos + NCHB) * CH, BLK)])

    nblk_out = (wpos >> 4) + nrem
    cntv[...] = jnp.broadcast_to(nblk_out, (16,))
    pltpu.sync_copy(cntv, counts.at[wid])


_partition = functools.partial(
    pl.kernel,
    out_type=[
        jax.ShapeDtypeStruct((NW * EPT,), jnp.int32),
        jax.ShapeDtypeStruct((NW * EPT,), jnp.int32),
        jax.ShapeDtypeStruct((NW * EPT,), jnp.float32),
        jax.ShapeDtypeStruct((NW, 16), jnp.int32),
    ],
    mesh=_mesh,
    scratch_types=[
        pltpu.VMEM((NCHB, CH), jnp.int32),
        pltpu.VMEM((NCHB, CH), jnp.int32),
        pltpu.VMEM((NCHB, CH), jnp.float32),
        pltpu.VMEM((STG,), jnp.int32),
        pltpu.VMEM((STG,), jnp.int32),
        pltpu.VMEM((STG,), jnp.float32),
        pltpu.VMEM((16,), jnp.int32),
        pltpu.SemaphoreType.DMA,
    ],
    compiler_params=_cparams,
)(_partition_body)


NSEG = 25
SLOTC = EPT // CH
STG2 = 416


def _colsplit_body(lrows_hbm, cols_hbm, vals_hbm, cntsA_hbm,
                   qrows, qcols, qvals, counts2,
                   rowsv, colsv, valsv, rstage, cstage, vstage, cntv, c2v):
    c = lax.axis_index("c")
    s = lax.axis_index("s")
    wid = s * NC + c

    pltpu.sync_copy(cntsA_hbm.at[wid], cntv)
    nblkA = cntv[...][0]

    iota = lax.broadcasted_iota(jnp.int32, (16,), 0)
    zero16 = jnp.zeros((16,), jnp.int32)

    def block_body(b, carry):
        offs, wpos = carry
        cb = wid * (EPT // CH) + b * NCHB

        pltpu.sync_copy(lrows_hbm.at[pl.ds(cb, NCHB)], rowsv)
        pltpu.sync_copy(cols_hbm.at[pl.ds(cb, NCHB)], colsv)
        pltpu.sync_copy(vals_hbm.at[pl.ds(cb, NCHB)], valsv)

        def win_body(w, carry2):
            offs2, wpos2 = carry2
            j = w

            def grp(g, offs3):
                gg = g * 16
                lr = rowsv[j, pl.ds(gg, 16)]
                cl = colsv[j, pl.ds(gg, 16)]
                v = valsv[j, pl.ds(gg, 16)]
                seg16 = lax.shift_right_logical(cl, 11)
                lcol = jnp.bitwise_and(cl, 2047)
                valid = lr < HALF
                new_offs = []
                for t in range(NSEG):
                    m = valid & (seg16 == t)
                    o = offs3[t]
                    plsc.store_compressed(rstage.at[t].at[pl.ds(o, 16)],
                                          lr, mask=m)
                    plsc.store_compressed(cstage.at[t].at[pl.ds(o, 16)],
                                          lcol, mask=m)
                    plsc.store_compressed(vstage.at[t].at[pl.ds(o, 16)],
                                          v, mask=m)
                    new_offs.append(
                        o + plsc.all_reduce_population_count(m)[0])
                return tuple(new_offs)

            offs2 = lax.fori_loop(0, 8, grp, offs2)

            new_offs, new_wpos = [], []
            for t in range(NSEG):
                o, wp = offs2[t], wpos2[t]
                full = o >= CH

                @pl.when(full)
                def _(t=t, wp=wp):
                    dst = ((wid * NSEG + t) * SLOTC + wp) * CH
                    pltpu.sync_copy(rstage.at[t].at[pl.ds(0, CH)],
                                    qrows.at[pl.ds(dst, CH)])
                    pltpu.sync_copy(cstage.at[t].at[pl.ds(0, CH)],
                                    qcols.at[pl.ds(dst, CH)])
                    pltpu.sync_copy(vstage.at[t].at[pl.ds(0, CH)],
                                    qvals.at[pl.ds(dst, CH)])

                    def mv(g, carry3):
                        sl_s = pl.ds(CH + g * 16, 16)
                        sl_d = pl.ds(g * 16, 16)
                        rstage[t, sl_d] = rstage[t, sl_s]
                        cstage[t, sl_d] = cstage[t, sl_s]
                        vstage[t, sl_d] = vstage[t, sl_s]
                        return carry3
                    lax.fori_loop(0, (STG2 - CH) // 16, mv, 0)

                new_offs.append(jnp.where(full, o - CH, o))
                new_wpos.append(jnp.where(full, wp + 1, wp))
            return (tuple(new_offs), tuple(new_wpos))

        return lax.fori_loop(0, NCHB, win_body, (offs, wpos))

    z25 = (jnp.int32(0),) * NSEG
    offs, wpos = lax.fori_loop(0, nblkA, block_body, (z25, z25))

    cnt_lanes = [zero16, zero16]
    for t in range(NSEG):
        o = offs[t]

        def fill(g, carry):
            pos = iota + g * 16
            m = pos >= o
            sl = pl.ds(g * 16, 16)
            rstage[t, sl] = jnp.where(m, jnp.int32(HALF), rstage[t, sl])
            cstage[t, sl] = jnp.where(m, zero16, cstage[t, sl])
            vstage[t, sl] = jnp.where(m, jnp.float32(0), vstage[t, sl])
            return carry

        lax.fori_loop(0, CH * 2 // 16, fill, 0)
        nrem = (o + CH - 1) >> 7

        @pl.when(nrem >= 1)
        def _(t=t, wp=wpos[t]):
            dst = ((wid * NSEG + t) * SLOTC + wp) * CH
            pltpu.sync_copy(rstage.at[t].at[pl.ds(0, CH)],
                            qrows.at[pl.ds(dst, CH)])
            pltpu.sync_copy(cstage.at[t].at[pl.ds(0, CH)],
                            qcols.at[pl.ds(dst, CH)])
            pltpu.sync_copy(vstage.at[t].at[pl.ds(0, CH)],
                            qvals.at[pl.ds(dst, CH)])

        @pl.when(nrem >= 2)
        def _(t=t, wp=wpos[t]):
            dst = ((wid * NSEG + t) * SLOTC + wp + 1) * CH
            pltpu.sync_copy(rstage.at[t].at[pl.ds(CH, CH)],
                            qrows.at[pl.ds(dst, CH)])
            pltpu.sync_copy(cstage.at[t].at[pl.ds(CH, CH)],
                            qcols.at[pl.ds(dst, CH)])
            pltpu.sync_copy(vstage.at[t].at[pl.ds(CH, CH)],
                            qvals.at[pl.ds(dst, CH)])

        cnt_t = wpos[t] + nrem
        c2v[...] = jnp.broadcast_to(cnt_t, (16,))
        pltpu.sync_copy(c2v, counts2.at[wid * 32 + t])


_colsplit = functools.partial(
    pl.kernel,
    out_type=[
        jax.ShapeDtypeStruct((NW * NSEG * SLOTC * CH,), jnp.int32),
        jax.ShapeDtypeStruct((NW * NSEG * SLOTC * CH,), jnp.int32),
        jax.ShapeDtypeStruct((NW * NSEG * SLOTC * CH,), jnp.float32),
        jax.ShapeDtypeStruct((NW * 32, 16), jnp.int32),
    ],
    mesh=_mesh,
    scratch_types=[
        pltpu.VMEM((NCHB, CH), jnp.int32),
        pltpu.VMEM((NCHB, CH), jnp.int32),
        pltpu.VMEM((NCHB, CH), jnp.float32),
        pltpu.VMEM((NSEG, STG2), jnp.int32),
        pltpu.VMEM((NSEG, STG2), jnp.int32),
        pltpu.VMEM((NSEG, STG2), jnp.float32),
        pltpu.VMEM((16,), jnp.int32),
        pltpu.VMEM((16,), jnp.int32),
    ],
    compiler_params=_cparams,
)(_colsplit_body)


def _spmm3_body(qrows_hbm, qcols_hbm, qvals_hbm, cnts2_hbm, x_hbm, out_hbm,
                acc, xstage, gbuf, rstg, cstg, vstg, cntv, gsem, ssem):
    c = lax.axis_index("c")
    s = lax.axis_index("s")
    wid = s * NC + c

    zvec = jnp.zeros((16,), jnp.float32)

    def zrow(i, carry):
        for jj in range(4):
            gbuf[0, i, pl.ds(jj * 16, 16)] = zvec
        return carry

    lax.fori_loop(0, CH, zrow, 0)
    rows_per_tile = ACC_ROWS // NS
    tbase = s * rows_per_tile
    for zi in range(rows_per_tile // CH):
        pltpu.sync_copy(gbuf.at[0], acc.at[pl.ds(tbase + zi * CH, CH)])
    rem = rows_per_tile % CH
    if rem:
        pltpu.sync_copy(gbuf.at[0].at[pl.ds(0, rem)],
                        acc.at[pl.ds(tbase + rows_per_tile - rem, rem)])

    plsc.subcore_barrier()

    def gstart(j, buf):
        pltpu.async_copy(xstage.at[cstg.at[j]], gbuf.at[buf], gsem.at[buf])

    def gwait(j, buf):
        pltpu.make_async_copy(xstage.at[cstg.at[j]], gbuf.at[buf],
                              gsem.at[buf]).wait()

    def sstart(j, buf):
        pltpu.async_copy(gbuf.at[buf], acc.at[rstg.at[j]], ssem.at[buf],
                         add=True)

    def swait(j, buf):
        pltpu.make_async_copy(gbuf.at[buf], acc.at[rstg.at[j]],
                              ssem.at[buf]).wait()

    def seg_body(seg, carry0):
        pltpu.sync_copy(x_hbm.at[pl.ds(seg * 2048 + s * CH, CH)],
                        xstage.at[pl.ds(s * CH, CH)])
        pltpu.sync_copy(cnts2_hbm.at[wid * 32 + seg], cntv)
        plsc.subcore_barrier()

        cnt = cntv[...][0]
        bbase = (wid * NSEG + seg) * SLOTC
        nstg = (cnt + 15) >> 4

        def stage_body(st, carry, cnt=cnt, bbase=bbase):
            pltpu.sync_copy(qrows_hbm.at[pl.ds(bbase + st * 16, 16)], rstg)
            pltpu.sync_copy(qcols_hbm.at[pl.ds(bbase + st * 16, 16)], cstg)
            pltpu.sync_copy(qvals_hbm.at[pl.ds(bbase + st * 16, 16)], vstg)
            nch = jnp.minimum(jnp.int32(16), cnt - st * 16)

            gstart(0, 0)

            def chunk_body(j, carry2):
                buf = jnp.bitwise_and(j, 1)
                gwait(j, buf)

                @pl.when(j + 1 < nch)
                def _prefetch():
                    @pl.when(j >= 1)
                    def _():
                        swait(j - 1, 1 - buf)
                    gstart(j + 1, 1 - buf)

                @plsc.parallel_loop(0, CH // 16, unroll=2)
                def scale_body(g):
                    e0 = g * 16
                    vals16 = vstg[j, pl.ds(e0, 16)]
                    for e2 in range(16):
                        e = e0 + e2
                        vsplat = jnp.broadcast_to(vals16[e2], (16,))
                        for kk in range(4):
                            sl = pl.ds(kk * 16, 16)
                            gbuf[buf, e, sl] = gbuf[buf, e, sl] * vsplat

                sstart(j, buf)
                return carry2

            lax.fori_loop(0, nch, chunk_body, 0)

            @pl.when(nch >= 2)
            def _():
                swait(nch - 2, jnp.bitwise_and(nch - 2, 1))
            swait(nch - 1, jnp.bitwise_and(nch - 1, 1))
            return carry

        lax.fori_loop(0, nstg, stage_body, 0)
        plsc.subcore_barrier()
        return carry0

    lax.fori_loop(0, NSEG, seg_body, 0)

    WB = 200
    NWB = HALF // WB

    def wb_body(w, carry):
        idx = s + w * NS

        @pl.when(idx < NWB)
        def _():
            pltpu.sync_copy(acc.at[pl.ds(idx * WB, WB)],
                            out_hbm.at[pl.ds(c * HALF + idx * WB, WB)])
        return carry

    lax.fori_loop(0, (NWB + NS - 1) // NS, wb_body, 0)


_spmm3 = functools.partial(
    pl.kernel,
    out_type=jax.ShapeDtypeStruct((N, D), jnp.float32),
    mesh=_mesh,
    scratch_types=[
        pltpu.VMEM_SHARED((ACC_ROWS, D), jnp.float32),
        pltpu.VMEM_SHARED((2048, D), jnp.float32),
        pltpu.VMEM((2, CH, D), jnp.float32),
        pltpu.VMEM((16, CH), jnp.int32),
        pltpu.VMEM((16, CH), jnp.int32),
        pltpu.VMEM((16, CH), jnp.float32),
        pltpu.VMEM((16,), jnp.int32),
        pltpu.SemaphoreType.DMA((2,)),
        pltpu.SemaphoreType.DMA((2,)),
    ],
    compiler_params=_cparams,
)(_spmm3_body)


def _spmm_body(lrows_hbm, cols_hbm, vals_hbm, cnts_hbm, x_hbm, out_hbm,
               acc, gbuf, lidxv, colsv, valsv, cntv, gsem, ssem):
    c = lax.axis_index("c")
    s = lax.axis_index("s")
    wid = s * NC + c

    zvec = jnp.zeros((16,), jnp.float32)

    def zrow(i, carry):
        for jj in range(4):
            gbuf[0, i, pl.ds(jj * 16, 16)] = zvec
        return carry

    lax.fori_loop(0, CH, zrow, 0)
    rows_per_tile = ACC_ROWS // NS
    tbase = s * rows_per_tile
    for zi in range(rows_per_tile // CH):
        pltpu.sync_copy(gbuf.at[0], acc.at[pl.ds(tbase + zi * CH, CH)])
    rem = rows_per_tile % CH
    if rem:
        pltpu.sync_copy(gbuf.at[0].at[pl.ds(0, rem)],
                        acc.at[pl.ds(tbase + rows_per_tile - rem, rem)])

    pltpu.sync_copy(cnts_hbm.at[wid], cntv)
    nblk = cntv[...][0]
    plsc.subcore_barrier()

    NB = 3

    def gstart(j, buf):
        pltpu.async_copy(x_hbm.at[colsv.at[j]], gbuf.at[buf], gsem.at[buf])

    def gwait(j, buf):
        pltpu.make_async_copy(x_hbm.at[colsv.at[j]], gbuf.at[buf],
                              gsem.at[buf]).wait()

    def sstart(j, buf):
        pltpu.async_copy(gbuf.at[buf], acc.at[lidxv.at[j]], ssem.at[buf],
                         add=True)

    def swait(j, buf):
        pltpu.make_async_copy(gbuf.at[buf], acc.at[lidxv.at[j]],
                              ssem.at[buf]).wait()

    def block_body(b, carry):
        run = b < nblk

        @pl.when(run)
        def _blk():
            cb = wid * (EPT // CH) + b * NCHB
            pltpu.sync_copy(lrows_hbm.at[pl.ds(cb, NCHB)], lidxv)
            pltpu.sync_copy(cols_hbm.at[pl.ds(cb, NCHB)], colsv)
            pltpu.sync_copy(vals_hbm.at[pl.ds(cb, NCHB)], valsv)

            gstart(0, 0)
            gstart(1, 1)

            def chunk_body(j, carry2):
                buf = jnp.remainder(j, NB)
                gwait(j, buf)

                def scale_body(g, carry3):
                    e0 = g * 16
                    vals16 = valsv[j, pl.ds(e0, 16)]
                    for e2 in range(16):
                        e = e0 + e2
                        vsplat = jnp.broadcast_to(vals16[e2], (16,))
                        for kk in range(4):
                            sl = pl.ds(kk * 16, 16)
                            gbuf[buf, e, sl] = gbuf[buf, e, sl] * vsplat
                    return carry3

                lax.fori_loop(0, CH // 16, scale_body, 0)
                sstart(j, buf)

                @pl.when(j + 2 < NCHB)
                def _prefetch():
                    nbuf = jnp.remainder(j + 2, NB)

                    @pl.when(j >= 1)
                    def _():
                        swait(j - 1, nbuf)
                    gstart(j + 2, nbuf)
                return carry2

            lax.fori_loop(0, NCHB, chunk_body, 0)
            swait(NCHB - 3, jnp.remainder(NCHB - 3, NB))
            swait(NCHB - 2, jnp.remainder(NCHB - 2, NB))
            swait(NCHB - 1, jnp.remainder(NCHB - 1, NB))
        return carry

    lax.fori_loop(0, NBLK, block_body, 0)
    plsc.subcore_barrier()

    WB = 200
    NWB = HALF // WB

    def wb_body(w, carry):
        idx = s + w * NS

        @pl.when(idx < NWB)
        def _():
            pltpu.sync_copy(acc.at[pl.ds(idx * WB, WB)],
                            out_hbm.at[pl.ds(c * HALF + idx * WB, WB)])
        return carry

    lax.fori_loop(0, (NWB + NS - 1) // NS, wb_body, 0)


_spmm = functools.partial(
    pl.kernel,
    out_type=jax.ShapeDtypeStruct((N, D), jnp.float32),
    mesh=_mesh,
    scratch_types=[
        pltpu.VMEM_SHARED((ACC_ROWS, D), jnp.float32),
        pltpu.VMEM((3, CH, D), jnp.float32),
        pltpu.VMEM((NCHB, CH), jnp.int32),
        pltpu.VMEM((NCHB, CH), jnp.int32),
        pltpu.VMEM((NCHB, CH), jnp.float32),
        pltpu.VMEM((16,), jnp.int32),
        pltpu.SemaphoreType.DMA((3,)),
        pltpu.SemaphoreType.DMA((3,)),
    ],
    compiler_params=_cparams,
)(_spmm_body)


def _dense_body(L_ref, e_ref, wg_ref, wb_ref, b_ref, act_ref, nrm_ref):
    l = L_ref[...]
    e = e_ref[...]
    x = (jnp.dot(l + e, wg_ref[...], preferred_element_type=jnp.float32)
         + jnp.dot(l * e, wb_ref[...], preferred_element_type=jnp.float32)
         + 2.0 * b_ref[...])
    act = jnp.where(x >= 0, x, 0.01 * x)
    act_ref[...] = act
    nrm = jnp.maximum(jnp.sqrt(jnp.sum(act * act, axis=1, keepdims=True)),
                      1e-12)
    nrm_ref[...] = act / nrm


_DENSE_R = 1000


def _dense(L, ego, Wg, Wb, b):
    return pl.pallas_call(
        _dense_body,
        grid=(N // _DENSE_R,),
        in_specs=[
            pl.BlockSpec((_DENSE_R, D), lambda i: (i, 0)),
            pl.BlockSpec((_DENSE_R, D), lambda i: (i, 0)),
            pl.BlockSpec((D, D), lambda i: (0, 0)),
            pl.BlockSpec((D, D), lambda i: (0, 0)),
            pl.BlockSpec((1, D), lambda i: (0, 0)),
        ],
        out_specs=[
            pl.BlockSpec((_DENSE_R, D), lambda i: (i, 0)),
            pl.BlockSpec((_DENSE_R, D), lambda i: (i, 0)),
        ],
        out_shape=[jax.ShapeDtypeStruct((N, D), jnp.float32)] * 2,
    )(L, ego, Wg, Wb, b)


GP = BATCH // (NC * NS)


def _final_gather_body(t0, t1, t2, t3, users, pos, neg, *rest):
    outs = rest[:12]
    idxv, rowsv, sem = rest[12:]
    wid = lax.axis_index("s") * NC + lax.axis_index("c")
    gbase = wid * GP
    tabs = (t0, t1, t2, t3)
    for bi, bidx in enumerate((users, pos, neg)):
        pltpu.sync_copy(bidx.at[pl.ds(gbase, GP)], idxv)
        if bi > 0:
            for jj in range(GP // 16):
                sl = pl.ds(jj * 16, 16)
                idxv[sl] = idxv[sl] + jnp.int32(N_USER)
        for ti in range(4):
            pltpu.async_copy(tabs[ti].at[idxv], rowsv, sem).wait()
            pltpu.sync_copy(rowsv, outs[bi * 4 + ti].at[pl.ds(gbase, GP)])


_final_gather = functools.partial(
    pl.kernel,
    out_type=[jax.ShapeDtypeStruct((BATCH, D), jnp.float32)] * 12,
    mesh=_mesh,
    scratch_types=[
        pltpu.VMEM((GP,), jnp.int32),
        pltpu.VMEM((GP, D), jnp.float32),
        pltpu.SemaphoreType.DMA,
    ],
    compiler_params=pltpu.CompilerParams(use_tc_tiling_on_sc=False),
)(_final_gather_body)


def kernel(user_emb, item_emb, W_gc_0, W_gc_1, W_gc_2, W_bi_0, W_bi_1, W_bi_2,
           b_bi_0, b_bi_1, b_bi_2, edge_index, edge_vals, users, pos_items,
           neg_items):
    ego0 = jnp.concatenate([user_emb, item_emb], axis=0)
    pad = E_PAD - E
    rows2d = jnp.concatenate(
        [edge_index[0], jnp.full((pad,), N, jnp.int32)]).reshape(E_PAD // CH, CH)
    cols2d = jnp.concatenate(
        [edge_index[1], jnp.zeros((pad,), jnp.int32)]).reshape(E_PAD // CH, CH)
    vals2d = jnp.concatenate(
        [edge_vals, jnp.zeros((pad,), jnp.float32)]).reshape(E_PAD // CH, CH)

    plr, plc, plv, cnts = _partition(rows2d, cols2d, vals2d)
    plr = plr.reshape(NW * EPT // CH, CH)
    plc = plc.reshape(NW * EPT // CH, CH)
    plv = plv.reshape(NW * EPT // CH, CH)

    qr, qc, qv, cnts2 = _colsplit(plr, plc, plv, cnts)
    qr = qr.reshape(NW * NSEG * SLOTC, CH)
    qc = qc.reshape(NW * NSEG * SLOTC, CH)
    qv = qv.reshape(NW * NSEG * SLOTC, CH)

    Ws_gc = (W_gc_0, W_gc_1, W_gc_2)
    Ws_bi = (W_bi_0, W_bi_1, W_bi_2)
    bs = (b_bi_0, b_bi_1, b_bi_2)

    xpad = jnp.zeros((NSEG * 2048 - N, D), jnp.float32)
    ego = ego0
    norms = []
    for k in range(3):
        L = _spmm3(qr, qc, qv, cnts2, jnp.concatenate([ego, xpad], axis=0))
        ego, nrm = _dense(L, ego, Ws_gc[k], Ws_bi[k], bs[k])
        norms.append(nrm)

    outs = _final_gather(ego0, norms[0], norms[1], norms[2],
                         users, pos_items, neg_items)
    u = jnp.concatenate(outs[0:4], axis=1)
    p = jnp.concatenate(outs[4:8], axis=1)
    ng = jnp.concatenate(outs[8:12], axis=1)
    return u, p, ng


# speedup vs baseline: 1.0917x; 1.0210x over previous
import functools

import jax
import jax.numpy as jnp
from jax import lax
from jax.experimental import pallas as pl
from jax.experimental.pallas import tpu as pltpu
from jax.experimental.pallas import tpu_sc as plsc

N_USER = 25000
N_ITEM = 25000
N = N_USER + N_ITEM
E = 800000
D = 64
BATCH = 1024

NC = 2
NS = 16
NW = NC * NS
HALF = N // NC
ACC_ROWS = 25008
CH = 128
BLK = 2048
NCHB = BLK // CH
EPT = 51200
NBLK = EPT // BLK
E_PAD = EPT * NS
STG = 4224

_mesh = plsc.VectorSubcoreMesh(core_axis_name="c", subcore_axis_name="s")
_cparams = pltpu.CompilerParams(use_tc_tiling_on_sc=False,
                                needs_layout_passes=False)


def _partition_body(rows_hbm, cols_hbm, vals_hbm,
                    plrows, pcols, pvals, counts,
                    rowsv, colsv, valsv, srow, scol, sval, cntv, sem):
    c = lax.axis_index("c")
    s = lax.axis_index("s")
    wid = s * NC + c
    base = c * jnp.int32(HALF)
    wslot = wid * (EPT // CH)

    def block_body(b, carry):
        off, wpos = carry
        cb = s * (EPT // CH) + b * NCHB
        pltpu.sync_copy(rows_hbm.at[pl.ds(cb, NCHB)], rowsv)
        pltpu.sync_copy(cols_hbm.at[pl.ds(cb, NCHB)], colsv)
        pltpu.sync_copy(vals_hbm.at[pl.ds(cb, NCHB)], valsv)

        def grp(g, off2):
            j = g >> 3
            jj = jnp.bitwise_and(g, 7) * 16
            r = rowsv[j, pl.ds(jj, 16)]
            cl = colsv[j, pl.ds(jj, 16)]
            v = valsv[j, pl.ds(jj, 16)]
            local = r - base
            msk = (local >= 0) & (local < HALF)
            plsc.store_compressed(srow.at[pl.ds(off2, 16)], local, mask=msk)
            plsc.store_compressed(scol.at[pl.ds(off2, 16)], cl, mask=msk)
            plsc.store_compressed(sval.at[pl.ds(off2, 16)], v, mask=msk)
            return off2 + plsc.all_reduce_population_count(msk)[0]

        off = lax.fori_loop(0, BLK // 16, grp, off)

        flush = off >= BLK

        @pl.when(flush)
        def _():
            pltpu.sync_copy(srow.at[pl.ds(0, BLK)],
                            plrows.at[pl.ds((wslot + wpos) * CH, BLK)])
            pltpu.sync_copy(scol.at[pl.ds(0, BLK)],
                            pcols.at[pl.ds((wslot + wpos) * CH, BLK)])
            pltpu.sync_copy(sval.at[pl.ds(0, BLK)],
                            pvals.at[pl.ds((wslot + wpos) * CH, BLK)])

            def mv(g, carry2):
                sl_src = pl.ds(BLK + g * 16, 16)
                sl_dst = pl.ds(g * 16, 16)
                srow[sl_dst] = srow[sl_src]
                scol[sl_dst] = scol[sl_src]
                sval[sl_dst] = sval[sl_src]
                return carry2
            lax.fori_loop(0, (STG - BLK) // 16, mv, 0)

        off = jnp.where(flush, off - BLK, off)
        wpos = jnp.where(flush, wpos + NCHB, wpos)
        return (off, wpos)

    off, wpos = lax.fori_loop(0, NBLK, block_body,
                              (jnp.int32(0), jnp.int32(0)))

    iota = lax.broadcasted_iota(jnp.int32, (16,), 0)

    def fill(g, carry):
        pos = iota + g * 16
        m = pos >= off
        sl = pl.ds(g * 16, 16)
        srow[sl] = jnp.where(m, jnp.int32(HALF), srow[sl])
        scol[sl] = jnp.where(m, jnp.int32(0), scol[sl])
        sval[sl] = jnp.where(m, jnp.float32(0), sval[sl])
        return carry

    lax.fori_loop(0, STG // 16, fill, 0)

    nrem = (off + BLK - 1) >> 11

    @pl.when(nrem >= 1)
    def _():
        pltpu.sync_copy(srow.at[pl.ds(0, BLK)],
                        plrows.at[pl.ds((wslot + wpos) * CH, BLK)])
        pltpu.sync_copy(scol.at[pl.ds(0, BLK)],
                        pcols.at[pl.ds((wslot + wpos) * CH, BLK)])
        pltpu.sync_copy(sval.at[pl.ds(0, BLK)],
                        pvals.at[pl.ds((wslot + wpos) * CH, BLK)])

    @pl.when(nrem >= 2)
    def _():
        pltpu.sync_copy(srow.at[pl.ds(BLK, BLK)],
                        plrows.at[pl.ds((wslot + wpos + NCHB) * CH, BLK)])
        pltpu.sync_copy(scol.at[pl.ds(BLK, BLK)],
                        pcols.at[pl.ds((wslot + wpos + NCHB) * CH, BLK)])
        pltpu.sync_copy(sval.at[pl.ds(BLK, BLK)],
                        pvals.at[pl.ds((wslot + wpos + NCHB) * CH, BLK)])

    nblk_out = (wpos >> 4) + nrem
    cntv[...] = jnp.broadcast_to(nblk_out, (16,))
    pltpu.sync_copy(cntv, counts.at[wid])


_partition = functools.partial(
    pl.kernel,
    out_type=[
        jax.ShapeDtypeStruct((NW * EPT,), jnp.int32),
        jax.ShapeDtypeStruct((NW * EPT,), jnp.int32),
        jax.ShapeDtypeStruct((NW * EPT,), jnp.float32),
        jax.ShapeDtypeStruct((NW, 16), jnp.int32),
    ],
    mesh=_mesh,
    scratch_types=[
        pltpu.VMEM((NCHB, CH), jnp.int32),
        pltpu.VMEM((NCHB, CH), jnp.int32),
        pltpu.VMEM((NCHB, CH), jnp.float32),
        pltpu.VMEM((STG,), jnp.int32),
        pltpu.VMEM((STG,), jnp.int32),
        pltpu.VMEM((STG,), jnp.float32),
        pltpu.VMEM((16,), jnp.int32),
        pltpu.SemaphoreType.DMA,
    ],
    compiler_params=_cparams,
)(_partition_body)


NSEG = 25
SLOTC = EPT // CH
STG2 = 416


def _colsplit_body(lrows_hbm, cols_hbm, vals_hbm, cntsA_hbm,
                   qrows, qcols, qvals, counts2,
                   rowsv, colsv, valsv, rstage, cstage, vstage, cntv, c2v):
    c = lax.axis_index("c")
    s = lax.axis_index("s")
    wid = s * NC + c

    pltpu.sync_copy(cntsA_hbm.at[wid], cntv)
    nblkA = cntv[...][0]

    iota = lax.broadcasted_iota(jnp.int32, (16,), 0)
    zero16 = jnp.zeros((16,), jnp.int32)

    def block_body(b, carry):
        offs, wpos = carry
        cb = wid * (EPT // CH) + b * NCHB

        pltpu.sync_copy(lrows_hbm.at[pl.ds(cb, NCHB)], rowsv)
        pltpu.sync_copy(cols_hbm.at[pl.ds(cb, NCHB)], colsv)
        pltpu.sync_copy(vals_hbm.at[pl.ds(cb, NCHB)], valsv)

        def win_body(w, carry2):
            offs2, wpos2 = carry2
            j = w

            def grp(g, offs3):
                gg = g * 16
                lr = rowsv[j, pl.ds(gg, 16)]
                cl = colsv[j, pl.ds(gg, 16)]
                v = valsv[j, pl.ds(gg, 16)]
                seg16 = lax.shift_right_logical(cl, 11)
                lcol = jnp.bitwise_and(cl, 2047)
                valid = lr < HALF
                new_offs = []
                for t in range(NSEG):
                    m = valid & (seg16 == t)
                    o = offs3[t]
                    plsc.store_compressed(rstage.at[t].at[pl.ds(o, 16)],
                                          lr, mask=m)
                    plsc.store_compressed(cstage.at[t].at[pl.ds(o, 16)],
                                          lcol, mask=m)
                    plsc.store_compressed(vstage.at[t].at[pl.ds(o, 16)],
                                          v, mask=m)
                    new_offs.append(
                        o + plsc.all_reduce_population_count(m)[0])
                return tuple(new_offs)

            offs2 = lax.fori_loop(0, 8, grp, offs2)

            new_offs, new_wpos = [], []
            for t in range(NSEG):
                o, wp = offs2[t], wpos2[t]
                full = o >= CH

                @pl.when(full)
                def _(t=t, wp=wp):
                    dst = ((wid * NSEG + t) * SLOTC + wp) * CH
                    pltpu.sync_copy(rstage.at[t].at[pl.ds(0, CH)],
                                    qrows.at[pl.ds(dst, CH)])
                    pltpu.sync_copy(cstage.at[t].at[pl.ds(0, CH)],
                                    qcols.at[pl.ds(dst, CH)])
                    pltpu.sync_copy(vstage.at[t].at[pl.ds(0, CH)],
                                    qvals.at[pl.ds(dst, CH)])

                    def mv(g, carry3):
                        sl_s = pl.ds(CH + g * 16, 16)
                        sl_d = pl.ds(g * 16, 16)
                        rstage[t, sl_d] = rstage[t, sl_s]
                        cstage[t, sl_d] = cstage[t, sl_s]
                        vstage[t, sl_d] = vstage[t, sl_s]
                        return carry3
                    lax.fori_loop(0, (STG2 - CH) // 16, mv, 0)

                new_offs.append(jnp.where(full, o - CH, o))
                new_wpos.append(jnp.where(full, wp + 1, wp))
            return (tuple(new_offs), tuple(new_wpos))

        return lax.fori_loop(0, NCHB, win_body, (offs, wpos))

    z25 = (jnp.int32(0),) * NSEG
    offs, wpos = lax.fori_loop(0, nblkA, block_body, (z25, z25))

    cnt_lanes = [zero16, zero16]
    for t in range(NSEG):
        o = offs[t]

        def fill(g, carry):
            pos = iota + g * 16
            m = pos >= o
            sl = pl.ds(g * 16, 16)
            rstage[t, sl] = jnp.where(m, jnp.int32(HALF), rstage[t, sl])
            cstage[t, sl] = jnp.where(m, zero16, cstage[t, sl])
            vstage[t, sl] = jnp.where(m, jnp.float32(0), vstage[t, sl])
            return carry

        lax.fori_loop(0, CH * 2 // 16, fill, 0)
        nrem = (o + CH - 1) >> 7

        @pl.when(nrem >= 1)
        def _(t=t, wp=wpos[t]):
            dst = ((wid * NSEG + t) * SLOTC + wp) * CH
            pltpu.sync_copy(rstage.at[t].at[pl.ds(0, CH)],
                            qrows.at[pl.ds(dst, CH)])
            pltpu.sync_copy(cstage.at[t].at[pl.ds(0, CH)],
                            qcols.at[pl.ds(dst, CH)])
            pltpu.sync_copy(vstage.at[t].at[pl.ds(0, CH)],
                            qvals.at[pl.ds(dst, CH)])

        @pl.when(nrem >= 2)
        def _(t=t, wp=wpos[t]):
            dst = ((wid * NSEG + t) * SLOTC + wp + 1) * CH
            pltpu.sync_copy(rstage.at[t].at[pl.ds(CH, CH)],
                            qrows.at[pl.ds(dst, CH)])
            pltpu.sync_copy(cstage.at[t].at[pl.ds(CH, CH)],
                            qcols.at[pl.ds(dst, CH)])
            pltpu.sync_copy(vstage.at[t].at[pl.ds(CH, CH)],
                            qvals.at[pl.ds(dst, CH)])

        cnt_t = wpos[t] + nrem
        c2v[...] = jnp.broadcast_to(cnt_t, (16,))
        pltpu.sync_copy(c2v, counts2.at[wid * 32 + t])


_colsplit = functools.partial(
    pl.kernel,
    out_type=[
        jax.ShapeDtypeStruct((NW * NSEG * SLOTC * CH,), jnp.int32),
        jax.ShapeDtypeStruct((NW * NSEG * SLOTC * CH,), jnp.int32),
        jax.ShapeDtypeStruct((NW * NSEG * SLOTC * CH,), jnp.float32),
        jax.ShapeDtypeStruct((NW * 32, 16), jnp.int32),
    ],
    mesh=_mesh,
    scratch_types=[
        pltpu.VMEM((NCHB, CH), jnp.int32),
        pltpu.VMEM((NCHB, CH), jnp.int32),
        pltpu.VMEM((NCHB, CH), jnp.float32),
        pltpu.VMEM((NSEG, STG2), jnp.int32),
        pltpu.VMEM((NSEG, STG2), jnp.int32),
        pltpu.VMEM((NSEG, STG2), jnp.float32),
        pltpu.VMEM((16,), jnp.int32),
        pltpu.VMEM((16,), jnp.int32),
    ],
    compiler_params=_cparams,
)(_colsplit_body)


def _spmm3_body(qrows_hbm, qcols_hbm, qvals_hbm, cnts2_hbm, x_hbm, out_hbm,
                acc, xstage, gbuf, rstg, cstg, vstg, cntv, gsem, ssem):
    c = lax.axis_index("c")
    s = lax.axis_index("s")
    wid = s * NC + c

    zvec = jnp.zeros((16,), jnp.float32)

    def zrow(i, carry):
        for jj in range(4):
            gbuf[0, i, pl.ds(jj * 16, 16)] = zvec
        return carry

    lax.fori_loop(0, CH, zrow, 0)
    rows_per_tile = ACC_ROWS // NS
    tbase = s * rows_per_tile
    for zi in range(rows_per_tile // CH):
        pltpu.sync_copy(gbuf.at[0], acc.at[pl.ds(tbase + zi * CH, CH)])
    rem = rows_per_tile % CH
    if rem:
        pltpu.sync_copy(gbuf.at[0].at[pl.ds(0, rem)],
                        acc.at[pl.ds(tbase + rows_per_tile - rem, rem)])

    plsc.subcore_barrier()

    def gstart(j, buf):
        pltpu.async_copy(xstage.at[cstg.at[j]], gbuf.at[buf], gsem.at[buf])

    def gwait(j, buf):
        pltpu.make_async_copy(xstage.at[cstg.at[j]], gbuf.at[buf],
                              gsem.at[buf]).wait()

    def sstart(j, buf):
        pltpu.async_copy(gbuf.at[buf], acc.at[rstg.at[j]], ssem.at[buf],
                         add=True)

    def swait(j, buf):
        pltpu.make_async_copy(gbuf.at[buf], acc.at[rstg.at[j]],
                              ssem.at[buf]).wait()

    def seg_body(seg, carry0):
        pltpu.sync_copy(x_hbm.at[pl.ds(seg * 2048 + s * CH, CH)],
                        xstage.at[pl.ds(s * CH, CH)])
        pltpu.sync_copy(cnts2_hbm.at[wid * 32 + seg], cntv)
        plsc.subcore_barrier()

        cnt = cntv[...][0]
        bbase = (wid * NSEG + seg) * SLOTC
        nstg = (cnt + 15) >> 4

        def stage_body(st, carry, cnt=cnt, bbase=bbase):
            pltpu.sync_copy(qrows_hbm.at[pl.ds(bbase + st * 16, 16)], rstg)
            pltpu.sync_copy(qcols_hbm.at[pl.ds(bbase + st * 16, 16)], cstg)
            pltpu.sync_copy(qvals_hbm.at[pl.ds(bbase + st * 16, 16)], vstg)
            nch = jnp.minimum(jnp.int32(16), cnt - st * 16)

            gstart(0, 0)

            def chunk_body(j, carry2):
                buf = jnp.bitwise_and(j, 1)
                gwait(j, buf)

                @pl.when(j + 1 < nch)
                def _prefetch():
                    @pl.when(j >= 1)
                    def _():
                        swait(j - 1, 1 - buf)
                    gstart(j + 1, 1 - buf)

                @plsc.parallel_loop(0, CH // 16, unroll=2)
                def scale_body(g):
                    e0 = g * 16
                    vals16 = vstg[j, pl.ds(e0, 16)]
                    for e2 in range(16):
                        e = e0 + e2
                        vsplat = jnp.broadcast_to(vals16[e2], (16,))
                        for kk in range(4):
                            sl = pl.ds(kk * 16, 16)
                            gbuf[buf, e, sl] = gbuf[buf, e, sl] * vsplat

                sstart(j, buf)
                return carry2

            lax.fori_loop(0, nch, chunk_body, 0)

            @pl.when(nch >= 2)
            def _():
                swait(nch - 2, jnp.bitwise_and(nch - 2, 1))
            swait(nch - 1, jnp.bitwise_and(nch - 1, 1))
            return carry

        lax.fori_loop(0, nstg, stage_body, 0)
        plsc.subcore_barrier()
        return carry0

    lax.fori_loop(0, NSEG, seg_body, 0)

    WB = 200
    NWB = HALF // WB

    def wb_body(w, carry):
        idx = s + w * NS

        @pl.when(idx < NWB)
        def _():
            pltpu.sync_copy(acc.at[pl.ds(idx * WB, WB)],
                            out_hbm.at[pl.ds(c * HALF + idx * WB, WB)])
        return carry

    lax.fori_loop(0, (NWB + NS - 1) // NS, wb_body, 0)


N_PAD = NSEG * 2048

_spmm3 = functools.partial(
    pl.kernel,
    out_type=jax.ShapeDtypeStruct((N_PAD, D), jnp.float32),
    mesh=_mesh,
    scratch_types=[
        pltpu.VMEM_SHARED((ACC_ROWS, D), jnp.float32),
        pltpu.VMEM_SHARED((2048, D), jnp.float32),
        pltpu.VMEM((2, CH, D), jnp.float32),
        pltpu.VMEM((16, CH), jnp.int32),
        pltpu.VMEM((16, CH), jnp.int32),
        pltpu.VMEM((16, CH), jnp.float32),
        pltpu.VMEM((16,), jnp.int32),
        pltpu.SemaphoreType.DMA((2,)),
        pltpu.SemaphoreType.DMA((2,)),
    ],
    compiler_params=_cparams,
)(_spmm3_body)


def _spmm_body(lrows_hbm, cols_hbm, vals_hbm, cnts_hbm, x_hbm, out_hbm,
               acc, gbuf, lidxv, colsv, valsv, cntv, gsem, ssem):
    c = lax.axis_index("c")
    s = lax.axis_index("s")
    wid = s * NC + c

    zvec = jnp.zeros((16,), jnp.float32)

    def zrow(i, carry):
        for jj in range(4):
            gbuf[0, i, pl.ds(jj * 16, 16)] = zvec
        return carry

    lax.fori_loop(0, CH, zrow, 0)
    rows_per_tile = ACC_ROWS // NS
    tbase = s * rows_per_tile
    for zi in range(rows_per_tile // CH):
        pltpu.sync_copy(gbuf.at[0], acc.at[pl.ds(tbase + zi * CH, CH)])
    rem = rows_per_tile % CH
    if rem:
        pltpu.sync_copy(gbuf.at[0].at[pl.ds(0, rem)],
                        acc.at[pl.ds(tbase + rows_per_tile - rem, rem)])

    pltpu.sync_copy(cnts_hbm.at[wid], cntv)
    nblk = cntv[...][0]
    plsc.subcore_barrier()

    NB = 3

    def gstart(j, buf):
        pltpu.async_copy(x_hbm.at[colsv.at[j]], gbuf.at[buf], gsem.at[buf])

    def gwait(j, buf):
        pltpu.make_async_copy(x_hbm.at[colsv.at[j]], gbuf.at[buf],
                              gsem.at[buf]).wait()

    def sstart(j, buf):
        pltpu.async_copy(gbuf.at[buf], acc.at[lidxv.at[j]], ssem.at[buf],
                         add=True)

    def swait(j, buf):
        pltpu.make_async_copy(gbuf.at[buf], acc.at[lidxv.at[j]],
                              ssem.at[buf]).wait()

    def block_body(b, carry):
        run = b < nblk

        @pl.when(run)
        def _blk():
            cb = wid * (EPT // CH) + b * NCHB
            pltpu.sync_copy(lrows_hbm.at[pl.ds(cb, NCHB)], lidxv)
            pltpu.sync_copy(cols_hbm.at[pl.ds(cb, NCHB)], colsv)
            pltpu.sync_copy(vals_hbm.at[pl.ds(cb, NCHB)], valsv)

            gstart(0, 0)
            gstart(1, 1)

            def chunk_body(j, carry2):
                buf = jnp.remainder(j, NB)
                gwait(j, buf)

                def scale_body(g, carry3):
                    e0 = g * 16
                    vals16 = valsv[j, pl.ds(e0, 16)]
                    for e2 in range(16):
                        e = e0 + e2
                        vsplat = jnp.broadcast_to(vals16[e2], (16,))
                        for kk in range(4):
                            sl = pl.ds(kk * 16, 16)
                            gbuf[buf, e, sl] = gbuf[buf, e, sl] * vsplat
                    return carry3

                lax.fori_loop(0, CH // 16, scale_body, 0)
                sstart(j, buf)

                @pl.when(j + 2 < NCHB)
                def _prefetch():
                    nbuf = jnp.remainder(j + 2, NB)

                    @pl.when(j >= 1)
                    def _():
                        swait(j - 1, nbuf)
                    gstart(j + 2, nbuf)
                return carry2

            lax.fori_loop(0, NCHB, chunk_body, 0)
            swait(NCHB - 3, jnp.remainder(NCHB - 3, NB))
            swait(NCHB - 2, jnp.remainder(NCHB - 2, NB))
            swait(NCHB - 1, jnp.remainder(NCHB - 1, NB))
        return carry

    lax.fori_loop(0, NBLK, block_body, 0)
    plsc.subcore_barrier()

    WB = 200
    NWB = HALF // WB

    def wb_body(w, carry):
        idx = s + w * NS

        @pl.when(idx < NWB)
        def _():
            pltpu.sync_copy(acc.at[pl.ds(idx * WB, WB)],
                            out_hbm.at[pl.ds(c * HALF + idx * WB, WB)])
        return carry

    lax.fori_loop(0, (NWB + NS - 1) // NS, wb_body, 0)


_spmm = functools.partial(
    pl.kernel,
    out_type=jax.ShapeDtypeStruct((N, D), jnp.float32),
    mesh=_mesh,
    scratch_types=[
        pltpu.VMEM_SHARED((ACC_ROWS, D), jnp.float32),
        pltpu.VMEM((3, CH, D), jnp.float32),
        pltpu.VMEM((NCHB, CH), jnp.int32),
        pltpu.VMEM((NCHB, CH), jnp.int32),
        pltpu.VMEM((NCHB, CH), jnp.float32),
        pltpu.VMEM((16,), jnp.int32),
        pltpu.SemaphoreType.DMA((3,)),
        pltpu.SemaphoreType.DMA((3,)),
    ],
    compiler_params=_cparams,
)(_spmm_body)


def _dense_body(L_ref, e_ref, wg_ref, wb_ref, b_ref, act_ref, nrm_ref):
    l = L_ref[...]
    e = e_ref[...]
    x = (jnp.dot(l + e, wg_ref[...], preferred_element_type=jnp.float32)
         + jnp.dot(l * e, wb_ref[...], preferred_element_type=jnp.float32)
         + 2.0 * b_ref[...])
    act = jnp.where(x >= 0, x, 0.01 * x)
    act_ref[...] = act
    nrm = jnp.maximum(jnp.sqrt(jnp.sum(act * act, axis=1, keepdims=True)),
                      1e-12)
    nrm_ref[...] = act / nrm


_DENSE_R = 1024


def _dense(L, ego, Wg, Wb, b):
    return pl.pallas_call(
        _dense_body,
        grid=(NSEG * 2048 // _DENSE_R,),
        in_specs=[
            pl.BlockSpec((_DENSE_R, D), lambda i: (i, 0)),
            pl.BlockSpec((_DENSE_R, D), lambda i: (i, 0)),
            pl.BlockSpec((D, D), lambda i: (0, 0)),
            pl.BlockSpec((D, D), lambda i: (0, 0)),
            pl.BlockSpec((1, D), lambda i: (0, 0)),
        ],
        out_specs=[
            pl.BlockSpec((_DENSE_R, D), lambda i: (i, 0)),
            pl.BlockSpec((_DENSE_R, D), lambda i: (i, 0)),
        ],
        out_shape=[jax.ShapeDtypeStruct((NSEG * 2048, D), jnp.float32)] * 2,
    )(L, ego, Wg, Wb, b)


GP = BATCH // (NC * NS)


def _final_gather_body(t0, t1, t2, t3, users, pos, neg, *rest):
    outs = rest[:12]
    idxv, rowsv, sem = rest[12:]
    wid = lax.axis_index("s") * NC + lax.axis_index("c")
    gbase = wid * GP
    tabs = (t0, t1, t2, t3)
    for bi, bidx in enumerate((users, pos, neg)):
        pltpu.sync_copy(bidx.at[pl.ds(gbase, GP)], idxv)
        if bi > 0:
            for jj in range(GP // 16):
                sl = pl.ds(jj * 16, 16)
                idxv[sl] = idxv[sl] + jnp.int32(N_USER)
        for ti in range(4):
            pltpu.async_copy(tabs[ti].at[idxv], rowsv, sem).wait()
            pltpu.sync_copy(rowsv, outs[bi * 4 + ti].at[pl.ds(gbase, GP)])


_final_gather = functools.partial(
    pl.kernel,
    out_type=[jax.ShapeDtypeStruct((BATCH, D), jnp.float32)] * 12,
    mesh=_mesh,
    scratch_types=[
        pltpu.VMEM((GP,), jnp.int32),
        pltpu.VMEM((GP, D), jnp.float32),
        pltpu.SemaphoreType.DMA,
    ],
    compiler_params=pltpu.CompilerParams(use_tc_tiling_on_sc=False),
)(_final_gather_body)


def kernel(user_emb, item_emb, W_gc_0, W_gc_1, W_gc_2, W_bi_0, W_bi_1, W_bi_2,
           b_bi_0, b_bi_1, b_bi_2, edge_index, edge_vals, users, pos_items,
           neg_items):
    ego0 = jnp.concatenate(
        [user_emb, item_emb,
         jnp.zeros((NSEG * 2048 - N, D), jnp.float32)], axis=0)
    pad = E_PAD - E
    rows2d = jnp.concatenate(
        [edge_index[0], jnp.full((pad,), N, jnp.int32)]).reshape(E_PAD // CH, CH)
    cols2d = jnp.concatenate(
        [edge_index[1], jnp.zeros((pad,), jnp.int32)]).reshape(E_PAD // CH, CH)
    vals2d = jnp.concatenate(
        [edge_vals, jnp.zeros((pad,), jnp.float32)]).reshape(E_PAD // CH, CH)

    plr, plc, plv, cnts = _partition(rows2d, cols2d, vals2d)
    plr = plr.reshape(NW * EPT // CH, CH)
    plc = plc.reshape(NW * EPT // CH, CH)
    plv = plv.reshape(NW * EPT // CH, CH)

    qr, qc, qv, cnts2 = _colsplit(plr, plc, plv, cnts)
    qr = qr.reshape(NW * NSEG * SLOTC, CH)
    qc = qc.reshape(NW * NSEG * SLOTC, CH)
    qv = qv.reshape(NW * NSEG * SLOTC, CH)

    Ws_gc = (W_gc_0, W_gc_1, W_gc_2)
    Ws_bi = (W_bi_0, W_bi_1, W_bi_2)
    bs = (b_bi_0, b_bi_1, b_bi_2)

    ego = ego0
    norms = []
    for k in range(3):
        L = _spmm3(qr, qc, qv, cnts2, ego)
        ego, nrm = _dense(L, ego, Ws_gc[k], Ws_bi[k], bs[k])
        norms.append(nrm)

    outs = _final_gather(ego0, norms[0], norms[1], norms[2],
                         users, pos_items, neg_items)
    u = jnp.concatenate(outs[0:4], axis=1)
    p = jnp.concatenate(outs[4:8], axis=1)
    ng = jnp.concatenate(outs[8:12], axis=1)
    return u, p, ng
